# baseline (speedup 1.0000x reference)
import functools

import jax
import jax.numpy as jnp
from jax import lax
from jax.experimental import pallas as pl
from jax.experimental.pallas import tpu as pltpu
from jax.experimental.pallas import tpu_sc as plsc

N = 10000
E = 320000
NC = 2
NS = 16
NW = NC * NS
RPT = N // NS
CH = 128
NCH = 80
EPW = NCH * CH
EPAD = NW * EPW - E
NBUF = 8
DEGW = 16

_mesh = plsc.VectorSubcoreMesh(core_axis_name="c", subcore_axis_name="s")


_SLAG = 4


def _deg_body(dst3_hbm, zeros_hbm, ones_hbm, degp_hbm, acc, dst_all, ones_v, ssem):
    c = lax.axis_index("c")
    s = lax.axis_index("s")
    rows0 = s * RPT
    w = s * NC + c
    d0 = pltpu.async_copy(zeros_hbm, acc.at[pl.ds(rows0, RPT)], ssem)
    d1 = pltpu.async_copy(ones_hbm, ones_v, ssem)
    d2 = pltpu.async_copy(dst3_hbm.at[w], dst_all, ssem)
    d0.wait()
    d1.wait()
    d2.wait()
    plsc.subcore_barrier()

    def body(j, carry):
        desc = pltpu.async_copy(ones_v, acc.at[dst_all.at[j]], ssem, add=True)

        @pl.when(j >= _SLAG)
        def _():
            desc.wait()

        return carry

    lax.fori_loop(0, NCH, body, 0)
    for _ in range(_SLAG):
        pltpu.make_async_copy(ones_v, acc.at[dst_all.at[0]], ssem).wait()
    plsc.subcore_barrier()
    pltpu.sync_copy(acc.at[pl.ds(rows0, RPT)],
                    degp_hbm.at[pl.ds(rows0, RPT), pl.ds(c * DEGW, DEGW)])


_sc_params = pltpu.CompilerParams(use_tc_tiling_on_sc=False)

_deg_kernel = pl.kernel(
    _deg_body,
    out_type=jax.ShapeDtypeStruct((N, 128), jnp.float32),
    mesh=_mesh,
    compiler_params=_sc_params,
    scratch_types=[
        pltpu.VMEM_SHARED((N + 8, DEGW), jnp.float32),
        pltpu.VMEM((NCH, CH), jnp.int32),
        pltpu.VMEM((CH, DEGW), jnp.float32),
        pltpu.SemaphoreType.DMA,
    ],
)


def _prop_body(d, g_hbm, src3_hbm, dst3_hbm, out_hbm, acc, src_all, dst_all,
               rows_refs, gsems, ssems):
    c = lax.axis_index("c")
    s = lax.axis_index("s")
    rows0 = s * RPT
    w = s * NC + c

    def _gather(j, b):
        pltpu.async_copy(g_hbm.at[src_all.at[j]], rows_refs[b], gsems[b])

    def _wait_gather(b):
        pltpu.make_async_copy(g_hbm.at[src_all.at[0]], rows_refs[b],
                              gsems[b]).wait()

    def _scatter(j, b):
        pltpu.async_copy(rows_refs[b], acc.at[dst_all.at[j]], ssems[b],
                         add=True)

    def _wait_scatter(b):
        pltpu.make_async_copy(rows_refs[b], acc.at[dst_all.at[0]],
                              ssems[b]).wait()

    pltpu.sync_copy(src3_hbm.at[w], src_all)
    pltpu.sync_copy(dst3_hbm.at[w], dst_all)
    for b in range(NBUF):
        _gather(b, b)
    pltpu.sync_copy(g_hbm.at[pl.ds(rows0, RPT)], acc.at[pl.ds(rows0, RPT)])
    plsc.subcore_barrier()

    def body(i, carry):
        for b in range(NBUF):
            _wait_gather(b)
            _scatter(i * NBUF + b, b)
        for b in range(NBUF):
            jn = i * NBUF + b + NBUF

            @pl.when(jn < NCH)
            def _(jn=jn, b=b):
                _wait_scatter(b)
                _gather(jn, b)

        return carry

    lax.fori_loop(0, NCH // NBUF, body, 0)
    for b in range(NBUF):
        _wait_scatter(b)
    plsc.subcore_barrier()
    pltpu.sync_copy(acc.at[pl.ds(rows0, RPT)],
                    out_hbm.at[pl.ds(rows0, RPT), pl.ds(c * d, d)])


def _make_prop(d):
    return pl.kernel(
        functools.partial(_prop_body, d),
        out_type=jax.ShapeDtypeStruct((N, 128), jnp.float32),
        mesh=_mesh,
        compiler_params=_sc_params,
        scratch_types=[
            pltpu.VMEM_SHARED((N + 8, d), jnp.float32),
            pltpu.VMEM((NCH, CH), jnp.int32),
            pltpu.VMEM((NCH, CH), jnp.int32),
            tuple(pltpu.VMEM((CH, d), jnp.float32) for _ in range(NBUF)),
            tuple(pltpu.SemaphoreType.DMA for _ in range(NBUF)),
            tuple(pltpu.SemaphoreType.DMA for _ in range(NBUF)),
        ],
    )


_prop64 = _make_prop(64)
_prop32 = _make_prop(32)


def _dense1_body(x_ref, w1_ref, degp_ref, g1_ref, dinv_ref):
    deg = degp_ref[:, 0:1] + degp_ref[:, DEGW:DEGW + 1] + 1.0
    dinv = lax.rsqrt(deg)
    dinv_ref[...] = dinv
    h1 = jnp.dot(x_ref[...], w1_ref[...], preferred_element_type=jnp.float32)
    g1_ref[...] = h1 * dinv


def _dense2_body(p1_ref, g1_ref, dinv_ref, b1_ref, w2_ref, g2_ref):
    dinv = dinv_ref[...]
    ssum = p1_ref[:, 0:64] + p1_ref[:, 64:128] - g1_ref[...]
    h = jnp.maximum(ssum * dinv + b1_ref[...], 0.0)
    g2_ref[...] = jnp.dot(h, w2_ref[...],
                          preferred_element_type=jnp.float32) * dinv


def _out_body(p2_ref, g2_ref, dinv_ref, b2_ref, z_ref):
    ssum = p2_ref[:, 0:32] + p2_ref[:, 32:64] - g2_ref[...]
    z_ref[...] = ssum * dinv_ref[...] + b2_ref[...]


_BLK = 2000


def _rows(bs):
    return pl.BlockSpec((_BLK, bs), lambda i: (i, 0))


def _full(r, c):
    return pl.BlockSpec((r, c), lambda i: (0, 0))


def kernel(x, edge_index, W1, b1, W2, b2):
    src3 = jnp.concatenate(
        [edge_index[0].astype(jnp.int32),
         jnp.zeros((EPAD,), jnp.int32)]).reshape(NW, NCH, CH)
    dst3 = jnp.concatenate(
        [edge_index[1].astype(jnp.int32),
         jnp.full((EPAD,), N, jnp.int32)]).reshape(NW, NCH, CH)
    zeros = jnp.zeros((RPT, DEGW), jnp.float32)
    ones = jnp.ones((CH, DEGW), jnp.float32)

    degp = _deg_kernel(dst3, zeros, ones)

    g1, dinv = pl.pallas_call(
        _dense1_body,
        grid=(N // _BLK,),
        in_specs=[_rows(128), _full(128, 64), _rows(128)],
        out_specs=[_rows(64), _rows(1)],
        out_shape=[
            jax.ShapeDtypeStruct((N, 64), jnp.float32),
            jax.ShapeDtypeStruct((N, 1), jnp.float32),
        ],
    )(x, W1, degp)

    p1 = _prop64(g1, src3, dst3)

    g2 = pl.pallas_call(
        _dense2_body,
        grid=(N // _BLK,),
        in_specs=[_rows(128), _rows(64), _rows(1), _full(1, 64), _full(64, 32)],
        out_specs=_rows(32),
        out_shape=jax.ShapeDtypeStruct((N, 32), jnp.float32),
    )(p1, g1, dinv, b1.reshape(1, 64), W2)

    p2 = _prop32(g2, src3, dst3)

    z = pl.pallas_call(
        _out_body,
        grid=(N // _BLK,),
        in_specs=[_rows(128), _rows(32), _rows(1), _full(1, 32)],
        out_specs=_rows(32),
        out_shape=jax.ShapeDtypeStruct((N, 32), jnp.float32),
    )(p2, g2, dinv, b2.reshape(1, 32))

    return z


# speedup vs baseline: 2.2257x; 2.2257x over previous
import functools

import jax
import jax.numpy as jnp
from jax import lax
from jax.experimental import pallas as pl
from jax.experimental.pallas import tpu as pltpu
from jax.experimental.pallas import tpu_sc as plsc

N = 10000
E = 320000
NC = 2
NS = 16
NW = NC * NS
EPW = E // NW
RPT = N // NS
CH = 125
NCH = EPW // CH
NBUF = 8
DEGW = 16

_mesh = plsc.VectorSubcoreMesh(core_axis_name="c", subcore_axis_name="s")


_SLAG = 4


def _deg_body(dst3_hbm, zeros_hbm, ones_hbm, degp_hbm, acc, dst_all, ones_v, ssem):
    c = lax.axis_index("c")
    s = lax.axis_index("s")
    rows0 = s * RPT
    w = s * NC + c
    d0 = pltpu.async_copy(zeros_hbm, acc.at[pl.ds(rows0, RPT)], ssem)
    d1 = pltpu.async_copy(ones_hbm, ones_v, ssem)
    d2 = pltpu.async_copy(dst3_hbm.at[w], dst_all, ssem)
    d0.wait()
    d1.wait()
    d2.wait()
    plsc.subcore_barrier()

    def body(j, carry):
        desc = pltpu.async_copy(ones_v, acc.at[dst_all.at[j]], ssem, add=True)

        @pl.when(j >= _SLAG)
        def _():
            desc.wait()

        return carry

    lax.fori_loop(0, NCH, body, 0)
    for _ in range(_SLAG):
        pltpu.make_async_copy(ones_v, acc.at[dst_all.at[0]], ssem).wait()
    plsc.subcore_barrier()
    pltpu.sync_copy(acc.at[pl.ds(rows0, RPT)],
                    degp_hbm.at[pl.ds(rows0, RPT), pl.ds(c * DEGW, DEGW)])


_sc_params = pltpu.CompilerParams(use_tc_tiling_on_sc=False)

_deg_kernel = pl.kernel(
    _deg_body,
    out_type=jax.ShapeDtypeStruct((N, 128), jnp.float32),
    mesh=_mesh,
    compiler_params=_sc_params,
    scratch_types=[
        pltpu.VMEM_SHARED((N, DEGW), jnp.float32),
        pltpu.VMEM((NCH, CH), jnp.int32),
        pltpu.VMEM((CH, DEGW), jnp.float32),
        pltpu.SemaphoreType.DMA,
    ],
)


def _prop_body(d, g_hbm, src3_hbm, dst3_hbm, out_hbm, acc, src_all, dst_all,
               rows_refs, gsems, ssems):
    c = lax.axis_index("c")
    s = lax.axis_index("s")
    rows0 = s * RPT
    w = s * NC + c

    def _gather(j, b):
        pltpu.async_copy(g_hbm.at[src_all.at[j]], rows_refs[b], gsems[b])

    def _wait_gather(b):
        pltpu.make_async_copy(g_hbm.at[src_all.at[0]], rows_refs[b],
                              gsems[b]).wait()

    def _scatter(j, b):
        pltpu.async_copy(rows_refs[b], acc.at[dst_all.at[j]], ssems[b],
                         add=True)

    def _wait_scatter(b):
        pltpu.make_async_copy(rows_refs[b], acc.at[dst_all.at[0]],
                              ssems[b]).wait()

    pltpu.sync_copy(src3_hbm.at[w], src_all)
    pltpu.sync_copy(dst3_hbm.at[w], dst_all)
    for b in range(NBUF):
        _gather(b, b)
    pltpu.sync_copy(g_hbm.at[pl.ds(rows0, RPT)], acc.at[pl.ds(rows0, RPT)])
    plsc.subcore_barrier()

    def body(i, carry):
        for b in range(NBUF):
            _wait_gather(b)
            _scatter(i * NBUF + b, b)
        for b in range(NBUF):
            jn = i * NBUF + b + NBUF

            @pl.when(jn < NCH)
            def _(jn=jn, b=b):
                _wait_scatter(b)
                _gather(jn, b)

        return carry

    lax.fori_loop(0, NCH // NBUF, body, 0)
    for b in range(NBUF):
        _wait_scatter(b)
    plsc.subcore_barrier()
    pltpu.sync_copy(acc.at[pl.ds(rows0, RPT)],
                    out_hbm.at[pl.ds(rows0, RPT), pl.ds(c * d, d)])


def _make_prop(d):
    return pl.kernel(
        functools.partial(_prop_body, d),
        out_type=jax.ShapeDtypeStruct((N, 128), jnp.float32),
        mesh=_mesh,
        compiler_params=_sc_params,
        scratch_types=[
            pltpu.VMEM_SHARED((N, d), jnp.float32),
            pltpu.VMEM((NCH, CH), jnp.int32),
            pltpu.VMEM((NCH, CH), jnp.int32),
            tuple(pltpu.VMEM((CH, d), jnp.float32) for _ in range(NBUF)),
            tuple(pltpu.SemaphoreType.DMA for _ in range(NBUF)),
            tuple(pltpu.SemaphoreType.DMA for _ in range(NBUF)),
        ],
    )


_prop64 = _make_prop(64)
_prop32 = _make_prop(32)


def _dense1_body(x_ref, w1_ref, degp_ref, g1_ref, dinv_ref):
    deg = degp_ref[:, 0:1] + degp_ref[:, DEGW:DEGW + 1] + 1.0
    dinv = lax.rsqrt(deg)
    dinv_ref[...] = dinv
    h1 = jnp.dot(x_ref[...], w1_ref[...], preferred_element_type=jnp.float32)
    g1_ref[...] = h1 * dinv


def _dense2_body(p1_ref, g1_ref, dinv_ref, b1_ref, w2_ref, g2_ref):
    dinv = dinv_ref[...]
    ssum = p1_ref[:, 0:64] + p1_ref[:, 64:128] - g1_ref[...]
    h = jnp.maximum(ssum * dinv + b1_ref[...], 0.0)
    g2_ref[...] = jnp.dot(h, w2_ref[...],
                          preferred_element_type=jnp.float32) * dinv


def _out_body(p2_ref, g2_ref, dinv_ref, b2_ref, z_ref):
    ssum = p2_ref[:, 0:32] + p2_ref[:, 32:64] - g2_ref[...]
    z_ref[...] = ssum * dinv_ref[...] + b2_ref[...]


_BLK = 2000


def _rows(bs):
    return pl.BlockSpec((_BLK, bs), lambda i: (i, 0))


def _full(r, c):
    return pl.BlockSpec((r, c), lambda i: (0, 0))


def kernel(x, edge_index, W1, b1, W2, b2):
    src3 = edge_index[0].astype(jnp.int32).reshape(NW, NCH, CH)
    dst3 = edge_index[1].astype(jnp.int32).reshape(NW, NCH, CH)
    zeros = jnp.zeros((RPT, DEGW), jnp.float32)
    ones = jnp.ones((CH, DEGW), jnp.float32)

    degp = _deg_kernel(dst3, zeros, ones)

    g1, dinv = pl.pallas_call(
        _dense1_body,
        grid=(N // _BLK,),
        in_specs=[_rows(128), _full(128, 64), _rows(128)],
        out_specs=[_rows(64), _rows(1)],
        out_shape=[
            jax.ShapeDtypeStruct((N, 64), jnp.float32),
            jax.ShapeDtypeStruct((N, 1), jnp.float32),
        ],
    )(x, W1, degp)

    p1 = _prop64(g1, src3, dst3)

    g2 = pl.pallas_call(
        _dense2_body,
        grid=(N // _BLK,),
        in_specs=[_rows(128), _rows(64), _rows(1), _full(1, 64), _full(64, 32)],
        out_specs=_rows(32),
        out_shape=jax.ShapeDtypeStruct((N, 32), jnp.float32),
    )(p1, g1, dinv, b1.reshape(1, 64), W2)

    p2 = _prop32(g2, src3, dst3)

    z = pl.pallas_call(
        _out_body,
        grid=(N // _BLK,),
        in_specs=[_rows(128), _rows(32), _rows(1), _full(1, 32)],
        out_specs=_rows(32),
        out_shape=jax.ShapeDtypeStruct((N, 32), jnp.float32),
    )(p2, g2, dinv, b2.reshape(1, 32))

    return z


# speedup vs baseline: 2.2779x; 1.0234x over previous
import functools

import jax
import jax.numpy as jnp
from jax import lax
from jax.experimental import pallas as pl
from jax.experimental.pallas import tpu as pltpu
from jax.experimental.pallas import tpu_sc as plsc

N = 10000
E = 320000
NC = 2
NS = 16
NW = NC * NS
EPW = E // NW
RPT = N // NS
CH = 125
NCH = EPW // CH
NBUF = 8
DEGW = 8

_mesh = plsc.VectorSubcoreMesh(core_axis_name="c", subcore_axis_name="s")


_SLAG = 4


def _deg_body(dst3_hbm, zeros_hbm, ones_hbm, degp_hbm, acc, dst_all, ones_v, ssem):
    c = lax.axis_index("c")
    s = lax.axis_index("s")
    rows0 = s * RPT
    w = s * NC + c
    d0 = pltpu.async_copy(zeros_hbm, acc.at[pl.ds(rows0, RPT)], ssem)
    d1 = pltpu.async_copy(ones_hbm, ones_v, ssem)
    d2 = pltpu.async_copy(dst3_hbm.at[w], dst_all, ssem)
    d0.wait()
    d1.wait()
    d2.wait()
    plsc.subcore_barrier()

    def body(j, carry):
        desc = pltpu.async_copy(ones_v, acc.at[dst_all.at[j]], ssem, add=True)

        @pl.when(j >= _SLAG)
        def _():
            desc.wait()

        return carry

    lax.fori_loop(0, NCH, body, 0)
    for _ in range(_SLAG):
        pltpu.make_async_copy(ones_v, acc.at[dst_all.at[0]], ssem).wait()
    plsc.subcore_barrier()
    pltpu.sync_copy(acc.at[pl.ds(rows0, RPT)],
                    degp_hbm.at[pl.ds(rows0, RPT), pl.ds(c * DEGW, DEGW)])


_sc_params = pltpu.CompilerParams(use_tc_tiling_on_sc=False)

_deg_kernel = pl.kernel(
    _deg_body,
    out_type=jax.ShapeDtypeStruct((N, 128), jnp.float32),
    mesh=_mesh,
    compiler_params=_sc_params,
    scratch_types=[
        pltpu.VMEM_SHARED((N, DEGW), jnp.float32),
        pltpu.VMEM((NCH, CH), jnp.int32),
        pltpu.VMEM((CH, DEGW), jnp.float32),
        pltpu.SemaphoreType.DMA,
    ],
)


def _prop_body(d, g_hbm, src3_hbm, dst3_hbm, out_hbm, acc, src_all, dst_all,
               rows_refs, gsems, ssems):
    c = lax.axis_index("c")
    s = lax.axis_index("s")
    rows0 = s * RPT
    w = s * NC + c

    def _gather(j, b):
        pltpu.async_copy(g_hbm.at[src_all.at[j]], rows_refs[b], gsems[b])

    def _wait_gather(b):
        pltpu.make_async_copy(g_hbm.at[src_all.at[0]], rows_refs[b],
                              gsems[b]).wait()

    def _scatter(j, b):
        pltpu.async_copy(rows_refs[b], acc.at[dst_all.at[j]], ssems[b],
                         add=True)

    def _wait_scatter(b):
        pltpu.make_async_copy(rows_refs[b], acc.at[dst_all.at[0]],
                              ssems[b]).wait()

    pltpu.sync_copy(src3_hbm.at[w], src_all)
    pltpu.sync_copy(dst3_hbm.at[w], dst_all)
    for b in range(NBUF):
        _gather(b, b)
    pltpu.sync_copy(g_hbm.at[pl.ds(rows0, RPT)], acc.at[pl.ds(rows0, RPT)])
    plsc.subcore_barrier()

    def body(i, carry):
        for b in range(NBUF):
            _wait_gather(b)
            _scatter(i * NBUF + b, b)
        for b in range(NBUF):
            jn = i * NBUF + b + NBUF

            @pl.when(jn < NCH)
            def _(jn=jn, b=b):
                _wait_scatter(b)
                _gather(jn, b)

        return carry

    lax.fori_loop(0, NCH // NBUF, body, 0)
    for b in range(NBUF):
        _wait_scatter(b)
    plsc.subcore_barrier()
    pltpu.sync_copy(acc.at[pl.ds(rows0, RPT)],
                    out_hbm.at[pl.ds(rows0, RPT), pl.ds(c * d, d)])


def _make_prop(d):
    return pl.kernel(
        functools.partial(_prop_body, d),
        out_type=jax.ShapeDtypeStruct((N, 128), jnp.float32),
        mesh=_mesh,
        compiler_params=_sc_params,
        scratch_types=[
            pltpu.VMEM_SHARED((N, d), jnp.float32),
            pltpu.VMEM((NCH, CH), jnp.int32),
            pltpu.VMEM((NCH, CH), jnp.int32),
            tuple(pltpu.VMEM((CH, d), jnp.float32) for _ in range(NBUF)),
            tuple(pltpu.SemaphoreType.DMA for _ in range(NBUF)),
            tuple(pltpu.SemaphoreType.DMA for _ in range(NBUF)),
        ],
    )


_prop64 = _make_prop(64)
_prop32 = _make_prop(32)


def _dense1_body(x_ref, w1_ref, degp_ref, g1_ref, dinv_ref):
    deg = degp_ref[:, 0:1] + degp_ref[:, DEGW:DEGW + 1] + 1.0
    dinv = lax.rsqrt(deg)
    dinv_ref[...] = dinv
    h1 = jnp.dot(x_ref[...], w1_ref[...], preferred_element_type=jnp.float32)
    g1_ref[...] = h1 * dinv


def _dense2_body(p1_ref, g1_ref, dinv_ref, b1_ref, w2_ref, g2_ref):
    dinv = dinv_ref[...]
    ssum = p1_ref[:, 0:64] + p1_ref[:, 64:128] - g1_ref[...]
    h = jnp.maximum(ssum * dinv + b1_ref[...], 0.0)
    g2_ref[...] = jnp.dot(h, w2_ref[...],
                          preferred_element_type=jnp.float32) * dinv


def _out_body(p2_ref, g2_ref, dinv_ref, b2_ref, z_ref):
    ssum = p2_ref[:, 0:32] + p2_ref[:, 32:64] - g2_ref[...]
    z_ref[...] = ssum * dinv_ref[...] + b2_ref[...]


_BLK = 2000


def _rows(bs):
    return pl.BlockSpec((_BLK, bs), lambda i: (i, 0))


def _full(r, c):
    return pl.BlockSpec((r, c), lambda i: (0, 0))


def kernel(x, edge_index, W1, b1, W2, b2):
    src3 = edge_index[0].astype(jnp.int32).reshape(NW, NCH, CH)
    dst3 = edge_index[1].astype(jnp.int32).reshape(NW, NCH, CH)
    zeros = jnp.zeros((RPT, DEGW), jnp.float32)
    ones = jnp.ones((CH, DEGW), jnp.float32)

    degp = _deg_kernel(dst3, zeros, ones)

    g1, dinv = pl.pallas_call(
        _dense1_body,
        grid=(N // _BLK,),
        in_specs=[_rows(128), _full(128, 64), _rows(128)],
        out_specs=[_rows(64), _rows(1)],
        out_shape=[
            jax.ShapeDtypeStruct((N, 64), jnp.float32),
            jax.ShapeDtypeStruct((N, 1), jnp.float32),
        ],
    )(x, W1, degp)

    p1 = _prop64(g1, src3, dst3)

    g2 = pl.pallas_call(
        _dense2_body,
        grid=(N // _BLK,),
        in_specs=[_rows(128), _rows(64), _rows(1), _full(1, 64), _full(64, 32)],
        out_specs=_rows(32),
        out_shape=jax.ShapeDtypeStruct((N, 32), jnp.float32),
    )(p1, g1, dinv, b1.reshape(1, 64), W2)

    p2 = _prop32(g2, src3, dst3)

    z = pl.pallas_call(
        _out_body,
        grid=(N // _BLK,),
        in_specs=[_rows(128), _rows(32), _rows(1), _full(1, 32)],
        out_specs=_rows(32),
        out_shape=jax.ShapeDtypeStruct((N, 32), jnp.float32),
    )(p2, g2, dinv, b2.reshape(1, 32))

    return z
